# baseline (speedup 1.0000x reference)
import functools

import jax
import jax.numpy as jnp
from jax import lax
from jax.experimental import pallas as pl
from jax.experimental.pallas import tpu as pltpu
from jax.experimental.pallas import tpu_sc as plsc

_B = 16384
_D = 32
_DP = 33
_ROWS = 1000000
_LB = 128
_NBLK = _ROWS // _LB
_TAIL = _ROWS - _NBLK * _LB
_INFO = plsc.get_sparse_core_info()
_NC = _INFO.num_cores
_NS = _INFO.num_subcores
_L = _INFO.num_lanes
_NW = _NC * _NS
_BPW = _B // _NW
_CHUNK = 128
_NCH = _BPW // _CHUNK
_GROUPS = _BPW // _L
_PAIRS = (_NBLK // _NW) // 2
_EXTRA = _NBLK - _PAIRS * 2 * _NW
_OUTW = _LB * _DP

_MESH = plsc.VectorSubcoreMesh(core_axis_name="c", subcore_axis_name="s")


def _rsqrt(x):
    i = plsc.bitcast(x, jnp.int32)
    i = 0x5F3759DF - lax.shift_right_logical(i, 1)
    y = plsc.bitcast(i, jnp.float32)
    for _ in range(3):
        y = y * (1.5 - 0.5 * x * y * y)
    return y


def _renorm_scale(sumsq):
    r = _rsqrt(sumsq)
    norm = sumsq * r
    inv = 1.0 / (norm + 1e-7)
    return jnp.where(norm > 1.0, inv, jnp.ones_like(norm))


@functools.partial(
    pl.kernel,
    mesh=_MESH,
    compiler_params=pltpu.CompilerParams(
        needs_layout_passes=False, use_tc_tiling_on_sc=True),
    out_type=(jax.ShapeDtypeStruct((_ROWS * _DP,), jnp.float32),
              jax.ShapeDtypeStruct((_ROWS * _DP,), jnp.float32)),
    scratch_types=[
        pltpu.VMEM((_D, _LB), jnp.float32),
        pltpu.VMEM((_D, _LB), jnp.float32),
        pltpu.VMEM((_OUTW,), jnp.float32),
        pltpu.VMEM((_OUTW,), jnp.float32),
        pltpu.SemaphoreType.DMA,
        pltpu.SemaphoreType.DMA,
        pltpu.SemaphoreType.DMA,
        pltpu.SemaphoreType.DMA,
    ],
)
def _transpose_kernel(utabt_hbm, itabt_hbm, tailu_hbm, tailv_hbm,
                      flatu_hbm, flatv_hbm,
                      ina_v, inb_v, outa_v, outb_v,
                      sia, sib, soa, sob):
    wid = lax.axis_index("s") * _NC + lax.axis_index("c")
    iota33 = lax.iota(jnp.int32, _L) * _DP

    def table(tabt_hbm, flat_hbm, tail_hbm):
        def fire_in(b, buf, sem):
            k = b * _NW + wid
            col0 = pl.multiple_of(k * _LB, _LB)
            pltpu.async_copy(tabt_hbm.at[:, pl.ds(col0, _LB)], buf, sem)

        def wait_in(buf, sem):
            pltpu.make_async_copy(tabt_hbm.at[:, pl.ds(0, _LB)], buf,
                                  sem).wait()

        def process(buf_in, buf_out):
            for f in range(_D):
                for c in range(_LB // _L):
                    v = buf_in[f, pl.ds(c * _L, _L)]
                    idx = iota33 + (c * _L * _DP + f)
                    plsc.store_scatter(buf_out, [idx], v)

        def fire_out(b, buf, sem):
            k = b * _NW + wid
            pltpu.async_copy(buf, flat_hbm.at[pl.ds(k * _OUTW, _OUTW)], sem)

        def wait_out(buf, sem):
            pltpu.make_async_copy(buf, flat_hbm.at[pl.ds(0, _OUTW)],
                                  sem).wait()

        fire_in(0, ina_v, sia)

        def body(i, carry):
            fire_in(2 * i + 1, inb_v, sib)
            wait_in(ina_v, sia)

            @pl.when(i > 0)
            def _():
                wait_out(outa_v, soa)

            process(ina_v, outa_v)
            fire_out(2 * i, outa_v, soa)

            @pl.when(i < _PAIRS - 1)
            def _():
                fire_in(2 * i + 2, ina_v, sia)

            wait_in(inb_v, sib)

            @pl.when(i > 0)
            def _():
                wait_out(outb_v, sob)

            process(inb_v, outb_v)
            fire_out(2 * i + 1, outb_v, sob)
            return carry

        lax.fori_loop(0, _PAIRS, body, 0)
        wait_out(outa_v, soa)
        wait_out(outb_v, sob)

        @pl.when(wid < _EXTRA)
        def _():
            b_extra = 2 * _PAIRS
            fire_in(b_extra, ina_v, sia)
            wait_in(ina_v, sia)
            process(ina_v, outa_v)
            fire_out(b_extra, outa_v, soa)
            wait_out(outa_v, soa)

        @pl.when(wid == _NW - 1)
        def _():
            pltpu.sync_copy(tail_hbm, outb_v.at[pl.ds(0, _TAIL * _DP)])
            pltpu.sync_copy(outb_v.at[pl.ds(0, _TAIL * _DP)],
                            flat_hbm.at[pl.ds(_NBLK * _OUTW, _TAIL * _DP)])

    table(utabt_hbm, flatu_hbm, tailu_hbm)
    table(itabt_hbm, flatv_hbm, tailv_hbm)


@functools.partial(
    pl.kernel,
    mesh=_MESH,
    compiler_params=pltpu.CompilerParams(
        needs_layout_passes=False, use_tc_tiling_on_sc=False),
    out_type=jax.ShapeDtypeStruct((_B,), jnp.float32),
    scratch_types=[
        pltpu.VMEM((_BPW,), jnp.int32),
        pltpu.VMEM((_BPW,), jnp.int32),
        pltpu.VMEM((_BPW, _DP), jnp.float32),
        pltpu.VMEM((_BPW, _DP), jnp.float32),
        pltpu.VMEM((_BPW,), jnp.float32),
        pltpu.SemaphoreType.DMA,
    ],
)
def _lookup_kernel(users_hbm, items_hbm, utab_hbm, itab_hbm, out_hbm,
                   uidx_v, iidx_v, urows_v, vrows_v, out_v, sem):
    wid = lax.axis_index("s") * _NC + lax.axis_index("c")
    base = wid * _BPW

    pltpu.sync_copy(users_hbm.at[pl.ds(base, _BPW)], uidx_v)
    pltpu.sync_copy(items_hbm.at[pl.ds(base, _BPW)], iidx_v)

    copies = []
    for c in range(_NCH):
        sl = pl.ds(c * _CHUNK, _CHUNK)
        copies.append(
            pltpu.async_copy(utab_hbm.at[uidx_v.at[sl]], urows_v.at[sl], sem))
        copies.append(
            pltpu.async_copy(itab_hbm.at[iidx_v.at[sl]], vrows_v.at[sl], sem))
    for cp in copies:
        cp.wait()

    lane = lax.iota(jnp.int32, _L)

    def group_body(g, carry):
        row0 = g * _L
        uu = jnp.zeros((_L,), jnp.float32)
        vv = jnp.zeros((_L,), jnp.float32)
        uv = jnp.zeros((_L,), jnp.float32)
        for e in range(_L):
            r = row0 + e
            u_lo = urows_v[r, pl.ds(0, _L)]
            u_hi = urows_v[r, pl.ds(_L, _L)]
            v_lo = vrows_v[r, pl.ds(0, _L)]
            v_hi = vrows_v[r, pl.ds(_L, _L)]
            p_uu = u_lo * u_lo + u_hi * u_hi
            p_vv = v_lo * v_lo + v_hi * v_hi
            p_uv = u_lo * v_lo + u_hi * v_hi
            m = lane == e
            uu = jnp.where(m, jnp.sum(p_uu), uu)
            vv = jnp.where(m, jnp.sum(p_vv), vv)
            uv = jnp.where(m, jnp.sum(p_uv), uv)
        su = _renorm_scale(uu)
        sv = _renorm_scale(vv)
        out_v[pl.ds(row0, _L)] = uv * su * sv
        return carry

    lax.fori_loop(0, _GROUPS, group_body, 0)

    pltpu.sync_copy(out_v, out_hbm.at[pl.ds(base, _BPW)])


def kernel(users, items, user_table, item_table):
    tailu = jnp.pad(user_table[_NBLK * _LB:],
                    ((0, 0), (0, _DP - _D))).reshape(_TAIL * _DP)
    tailv = jnp.pad(item_table[_NBLK * _LB:],
                    ((0, 0), (0, _DP - _D))).reshape(_TAIL * _DP)
    flat_u, flat_v = _transpose_kernel(user_table.T, item_table.T,
                                       tailu, tailv)
    return _lookup_kernel(users.astype(jnp.int32), items.astype(jnp.int32),
                          flat_u.reshape(_ROWS, _DP),
                          flat_v.reshape(_ROWS, _DP))


# speedup vs baseline: 8.6012x; 8.6012x over previous
import functools

import jax
import jax.numpy as jnp
from jax import lax
from jax.experimental import pallas as pl
from jax.experimental.pallas import tpu as pltpu
from jax.experimental.pallas import tpu_sc as plsc

_B = 16384
_D = 32
_ROWS = 1000000
_SUB = 8
_NT = _ROWS // _SUB
_INFO = plsc.get_sparse_core_info()
_NC = _INFO.num_cores
_NS = _INFO.num_subcores
_L = _INFO.num_lanes
_NW = _NC * _NS
_BPW = _B // _NW
_C = _L
_NCHUNK = _BPW // _C


def _rsqrt(x):
    i = plsc.bitcast(x, jnp.int32)
    i = 0x5F3759DF - lax.shift_right_logical(i, 1)
    y = plsc.bitcast(i, jnp.float32)
    for _ in range(3):
        y = y * (1.5 - 0.5 * x * y * y)
    return y


def _renorm_scale(sumsq):
    r = _rsqrt(sumsq)
    norm = sumsq * r
    inv = 1.0 / (norm + 1e-7)
    return jnp.where(norm > 1.0, inv, jnp.ones_like(norm))


_MESH = plsc.VectorSubcoreMesh(core_axis_name="c", subcore_axis_name="s")


@functools.partial(
    pl.kernel,
    mesh=_MESH,
    compiler_params=pltpu.CompilerParams(
        needs_layout_passes=False, use_tc_tiling_on_sc=True),
    out_type=jax.ShapeDtypeStruct((_B,), jnp.float32),
    scratch_types=[
        pltpu.VMEM((_BPW,), jnp.int32),
        pltpu.VMEM((_BPW,), jnp.int32),
        pltpu.VMEM((_C, _SUB, _D), jnp.float32),
        pltpu.VMEM((_C, _SUB, _D), jnp.float32),
        pltpu.VMEM((_C, _SUB, _D), jnp.float32),
        pltpu.VMEM((_C, _SUB, _D), jnp.float32),
        pltpu.VMEM((_BPW,), jnp.float32),
        pltpu.SemaphoreType.DMA,
        pltpu.SemaphoreType.DMA,
    ],
)
def _mf_kernel(users_hbm, items_hbm, utab_hbm, itab_hbm, out_hbm,
               uidx_s, iidx_s,
               au_v, av_v, bu_v, bv_v, out_v, sem_a, sem_b):
    wid = lax.axis_index("s") * _NC + lax.axis_index("c")
    base = wid * _BPW
    lane = lax.iota(jnp.int32, _L)

    pltpu.sync_copy(users_hbm.at[pl.ds(base, _BPW)], uidx_s)
    pltpu.sync_copy(items_hbm.at[pl.ds(base, _BPW)], iidx_s)

    def fire(c, bufu, bufv, sem):
        ex0 = c * _C
        tu_vec = lax.shift_right_logical(uidx_s[pl.ds(ex0, _L)], 3)
        ti_vec = lax.shift_right_logical(iidx_s[pl.ds(ex0, _L)], 3)
        for e in range(_C):
            pltpu.async_copy(utab_hbm.at[pl.ds(tu_vec[e], 1)],
                             bufu.at[pl.ds(e, 1)], sem)
            pltpu.async_copy(itab_hbm.at[pl.ds(ti_vec[e], 1)],
                             bufv.at[pl.ds(e, 1)], sem)

    def wait_pair(bufu, bufv, sem):
        pltpu.make_async_copy(utab_hbm.at[pl.ds(0, _C)], bufu, sem).wait()
        pltpu.make_async_copy(itab_hbm.at[pl.ds(0, _C)], bufv, sem).wait()

    def compute(c, bufu, bufv):
        ex0 = c * _C
        uu = jnp.zeros((_L,), jnp.float32)
        vv = jnp.zeros((_L,), jnp.float32)
        uv = jnp.zeros((_L,), jnp.float32)
        su_vec = uidx_s[pl.ds(ex0, _L)] & 7
        sv_vec = iidx_s[pl.ds(ex0, _L)] & 7
        for e in range(_C):
            su = su_vec[e]
            sv = sv_vec[e]
            u_lo = bufu[e, su, pl.ds(0, _L)]
            u_hi = bufu[e, su, pl.ds(_L, _L)]
            v_lo = bufv[e, sv, pl.ds(0, _L)]
            v_hi = bufv[e, sv, pl.ds(_L, _L)]
            p_uu = u_lo * u_lo + u_hi * u_hi
            p_vv = v_lo * v_lo + v_hi * v_hi
            p_uv = u_lo * v_lo + u_hi * v_hi
            m = lane == e
            uu = jnp.where(m, jnp.sum(p_uu), uu)
            vv = jnp.where(m, jnp.sum(p_vv), vv)
            uv = jnp.where(m, jnp.sum(p_uv), uv)
        scale = _renorm_scale(uu) * _renorm_scale(vv)
        out_v[pl.ds(ex0, _L)] = uv * scale

    fire(0, au_v, av_v, sem_a)

    def body(i, carry):
        c0 = 2 * i
        fire(c0 + 1, bu_v, bv_v, sem_b)
        wait_pair(au_v, av_v, sem_a)
        compute(c0, au_v, av_v)

        @pl.when(i < _NCHUNK // 2 - 1)
        def _():
            fire(c0 + 2, au_v, av_v, sem_a)

        wait_pair(bu_v, bv_v, sem_b)
        compute(c0 + 1, bu_v, bv_v)
        return carry

    lax.fori_loop(0, _NCHUNK // 2, body, 0)

    pltpu.sync_copy(out_v, out_hbm.at[pl.ds(base, _BPW)])


def kernel(users, items, user_table, item_table):
    utab3 = user_table.reshape(_NT, _SUB, _D)
    itab3 = item_table.reshape(_NT, _SUB, _D)
    return _mf_kernel(users.astype(jnp.int32), items.astype(jnp.int32),
                      utab3, itab3)
